# Initial kernel scaffold; baseline (speedup 1.0000x reference)
import jax
import jax.numpy as jnp
from jax.experimental import pallas as pl


def kernel(x):
    raise NotImplementedError("write your pallas kernel here")


# speedup vs baseline: 1.2114x; 1.2114x over previous
import functools

import jax
import jax.numpy as jnp
from jax import lax
from jax.experimental import pallas as pl
from jax.experimental.pallas import tpu as pltpu
from jax.experimental.pallas import tpu_sc as plsc

ROWS = 4096
COLS = 2048
NC = 2
NS = 16
LANES = 16
CPW = 128
STRIPES = COLS // CPW
G = CPW // LANES
CHUNK = 512
NCH = ROWS // CHUNK

_mesh = plsc.VectorSubcoreMesh(
    core_axis_name="c", subcore_axis_name="s", num_cores=NC, num_subcores=NS
)


@functools.partial(
    pl.kernel,
    out_type=jax.ShapeDtypeStruct((ROWS, COLS), jnp.float32),
    mesh=_mesh,
    scratch_types=[pltpu.VMEM((CHUNK, CPW), jnp.float32)],
)
def _cumsum_sc(x_hbm, out_hbm, buf):
    wid = lax.axis_index("s") * NC + lax.axis_index("c")

    @pl.when(wid < STRIPES)
    def _():
        col0 = wid * CPW

        def chunk_body(ci, carry):
            row0 = ci * CHUNK
            pltpu.sync_copy(x_hbm.at[pl.ds(row0, CHUNK), pl.ds(col0, CPW)], buf)

            def row_body(r, carry):
                new = []
                for g in range(G):
                    v = carry[g] + buf[r, pl.ds(g * LANES, LANES)]
                    buf[r, pl.ds(g * LANES, LANES)] = v
                    new.append(v)
                return tuple(new)

            carry = lax.fori_loop(0, CHUNK, row_body, carry)
            pltpu.sync_copy(buf, out_hbm.at[pl.ds(row0, CHUNK), pl.ds(col0, CPW)])
            return carry

        zero = jnp.zeros((LANES,), jnp.float32)
        lax.fori_loop(0, NCH, chunk_body, (zero,) * G)


def kernel(x):
    return _cumsum_sc(x)


# speedup vs baseline: 1.7613x; 1.4539x over previous
import functools

import jax
import jax.numpy as jnp
from jax import lax
from jax.experimental import pallas as pl
from jax.experimental.pallas import tpu as pltpu
from jax.experimental.pallas import tpu_sc as plsc

ROWS = 4096
COLS = 2048
NC = 2
NS = 16
LANES = 16
CPW = 128
STRIPES = COLS // CPW
G = CPW // LANES
CHUNK = 256
NCH = ROWS // CHUNK
NBUF = 3

_mesh = plsc.VectorSubcoreMesh(
    core_axis_name="c", subcore_axis_name="s", num_cores=NC, num_subcores=NS
)


@functools.partial(
    pl.kernel,
    out_type=jax.ShapeDtypeStruct((ROWS, COLS), jnp.float32),
    mesh=_mesh,
    scratch_types=[pltpu.VMEM((CHUNK, CPW), jnp.float32)] * NBUF
    + [pltpu.SemaphoreType.DMA] * (2 * NBUF),
)
def _cumsum_sc(x_hbm, out_hbm, b0, b1, b2, si0, si1, si2, so0, so1, so2):
    wid = lax.axis_index("s") * NC + lax.axis_index("c")

    @pl.when(wid < STRIPES)
    def _():
        col0 = wid * CPW
        bufs = (b0, b1, b2)
        sin = (si0, si1, si2)
        sout = (so0, so1, so2)

        def start_in(ci):
            b = ci % NBUF
            return pltpu.async_copy(
                x_hbm.at[pl.ds(ci * CHUNK, CHUNK), pl.ds(col0, CPW)],
                bufs[b], sin[b])

        def start_out(ci):
            b = ci % NBUF
            return pltpu.async_copy(
                bufs[b],
                out_hbm.at[pl.ds(ci * CHUNK, CHUNK), pl.ds(col0, CPW)],
                sout[b])

        in_h = {0: start_in(0), 1: start_in(1)}
        out_h = {}
        carry = (jnp.zeros((LANES,), jnp.float32),) * G
        for ci in range(NCH):
            in_h[ci].wait()
            buf = bufs[ci % NBUF]

            def row_body(r, carry, buf=buf):
                new = []
                for g in range(G):
                    v = carry[g] + buf[r, pl.ds(g * LANES, LANES)]
                    buf[r, pl.ds(g * LANES, LANES)] = v
                    new.append(v)
                return tuple(new)

            carry = lax.fori_loop(0, CHUNK, row_body, carry)
            out_h[ci] = start_out(ci)
            nx = ci + 2
            if nx < NCH:
                if nx - NBUF in out_h:
                    out_h.pop(nx - NBUF).wait()
                in_h[nx] = start_in(nx)
        for ci in sorted(out_h):
            out_h.pop(ci).wait()


def kernel(x):
    return _cumsum_sc(x)
